# baseline (speedup 1.0000x reference)
import jax
import jax.numpy as jnp
from jax.experimental import pallas as pl

_G = 76
_GG = _G * _G
_A = 3
_C = 80
_STRIDE = 8.0
_AW = (10.0, 16.0, 33.0)
_AH = (13.0, 30.0, 23.0)


def _decode_body(x_ref, bbox_ref, conf_ref, cls_ref):
    blk = x_ref[0]
    a = pl.program_id(0) % _A

    i_vec = jax.lax.broadcasted_iota(jnp.int32, (1, _GG), 1)
    gx = (i_vec % _G).astype(jnp.float32)
    gy = (i_vec // _G).astype(jnp.float32)
    aw = jnp.where(a == 0, _AW[0], jnp.where(a == 1, _AW[1], _AW[2]))
    ah = jnp.where(a == 0, _AH[0], jnp.where(a == 1, _AH[1], _AH[2]))

    xy = jax.nn.sigmoid(blk[0:2, :])
    bx = (xy[0:1] + gx) * _STRIDE
    by = (xy[1:2] + gy) * _STRIDE
    wh = jnp.exp(blk[2:4, :])
    bw = wh[0:1] * aw
    bh = wh[1:2] * ah
    bbox = jnp.concatenate([bx, by, bw, bh], axis=0)
    bbox_ref[0] = bbox.T

    conf_ref[0] = jax.nn.sigmoid(blk[4:5, :])
    cls_ref[0] = jax.nn.sigmoid(blk[5:, :]).T


@jax.jit
def kernel(x):
    B = x.shape[0]
    n = B * _A
    xr = x.reshape(n, _C + 5, _GG)
    bbox, conf, cls = pl.pallas_call(
        _decode_body,
        grid=(n,),
        in_specs=[pl.BlockSpec((1, _C + 5, _GG), lambda i: (i, 0, 0))],
        out_specs=[
            pl.BlockSpec((1, _GG, 4), lambda i: (i, 0, 0)),
            pl.BlockSpec((1, 1, _GG), lambda i: (i, 0, 0)),
            pl.BlockSpec((1, _GG, _C), lambda i: (i, 0, 0)),
        ],
        out_shape=[
            jax.ShapeDtypeStruct((n, _GG, 4), x.dtype),
            jax.ShapeDtypeStruct((n, 1, _GG), x.dtype),
            jax.ShapeDtypeStruct((n, _GG, _C), x.dtype),
        ],
    )(xr)
    return (
        bbox.reshape(B, _A, _G, _G, 4),
        conf.reshape(B, _A, _G, _G),
        cls.reshape(B, _A, _G, _G, _C),
    )


# speedup vs baseline: 1.7535x; 1.7535x over previous
import jax
import jax.numpy as jnp
from jax.experimental import pallas as pl

_G = 76
_A = 3
_C = 80
_STRIDE = 8.0
_AW = (10.0, 16.0, 33.0)
_AH = (13.0, 30.0, 23.0)


def _chan_to_minor(t):
    return jnp.transpose(jnp.transpose(t, (1, 0, 2)), (0, 2, 1))


def _decode_body(x_ref, bbox_ref, conf_ref, cls_ref):
    blk = x_ref[0]
    a = pl.program_id(1)

    gx = jax.lax.broadcasted_iota(jnp.int32, (_G, _G), 1).astype(jnp.float32)
    gy = jax.lax.broadcasted_iota(jnp.int32, (_G, _G), 0).astype(jnp.float32)
    aw = jnp.where(a == 0, _AW[0], jnp.where(a == 1, _AW[1], _AW[2]))
    ah = jnp.where(a == 0, _AH[0], jnp.where(a == 1, _AH[1], _AH[2]))

    bx = (jax.nn.sigmoid(blk[0]) + gx) * _STRIDE
    by = (jax.nn.sigmoid(blk[1]) + gy) * _STRIDE
    bw = jnp.exp(blk[2]) * aw
    bh = jnp.exp(blk[3]) * ah
    bbox = jnp.stack([bx, by, bw, bh], axis=0)
    bbox_ref[0, 0] = _chan_to_minor(bbox)

    conf_ref[0, 0] = jax.nn.sigmoid(blk[4])
    cls_ref[0, 0] = _chan_to_minor(jax.nn.sigmoid(blk[5:]))


@jax.jit
def kernel(x):
    B = x.shape[0]
    bbox, conf, cls = pl.pallas_call(
        _decode_body,
        grid=(B, _A),
        in_specs=[
            pl.BlockSpec((1, _C + 5, _G, _G), lambda i, j: (i, j, 0, 0)),
        ],
        out_specs=[
            pl.BlockSpec((1, 1, _G, _G, 4), lambda i, j: (i, j, 0, 0, 0)),
            pl.BlockSpec((1, 1, _G, _G), lambda i, j: (i, j, 0, 0)),
            pl.BlockSpec((1, 1, _G, _G, _C), lambda i, j: (i, j, 0, 0, 0)),
        ],
        out_shape=[
            jax.ShapeDtypeStruct((B, _A, _G, _G, 4), x.dtype),
            jax.ShapeDtypeStruct((B, _A, _G, _G), x.dtype),
            jax.ShapeDtypeStruct((B, _A, _G, _G, _C), x.dtype),
        ],
    )(x)
    return (bbox, conf, cls)
